# baseline (speedup 1.0000x reference)
import functools

import jax
import jax.numpy as jnp
from jax import lax
from jax.experimental import pallas as pl
from jax.experimental.pallas import tpu as pltpu
from jax.experimental.pallas import tpu_sc as plsc

VOCAB = 1000
BATCH, TLEN = 1024, 50
NC, NS = 1, 16
NW = NC * NS
N_TOK = BATCH * TLEN
B_PER_W = BATCH // NW
PER_W = B_PER_W * TLEN
N_PAIR = B_PER_W // 2
LANES = 16
PAD_W = PER_W + 64


def _lse_body(tbl_ref, lse_ref):
    l = tbl_ref[...]
    m = jnp.max(l, axis=1, keepdims=True)
    lse_ref[...] = m + jnp.log(jnp.sum(jnp.exp(l - m), axis=1, keepdims=True))


def _tc_lse(table):
    return pl.pallas_call(
        _lse_body,
        out_shape=jax.ShapeDtypeStruct((VOCAB, 1), jnp.float32),
    )(table)


def _gather_body(idx_hbm, idx3_hbm, tgt_hbm, lse_hbm, table_hbm,
                 out_hbm, part_hbm,
                 idx_v, idx2_v, tgt_v, lse_v, part_v, buf0, buf1, sem0, sem1):
    wid = lax.axis_index("s") * NC + lax.axis_index("c")
    base = wid * PER_W
    bbase = wid * B_PER_W
    pltpu.sync_copy(idx_hbm.at[pl.ds(base, PER_W)], idx_v.at[pl.ds(0, PER_W)])
    pltpu.sync_copy(tgt_hbm.at[pl.ds(base, PER_W)], tgt_v.at[pl.ds(0, PER_W)])
    pltpu.sync_copy(idx3_hbm.at[wid], idx2_v)
    pltpu.sync_copy(lse_hbm, lse_v)

    def start(c, buf, sem):
        pltpu.async_copy(table_hbm.at[idx2_v.at[c]], buf, sem)

    def wait(buf, sem):
        pltpu.make_async_copy(table_hbm.at[idx2_v.at[0]], buf, sem).wait()

    lane = lax.iota(jnp.int32, LANES)
    full = lane < LANES

    def chunk_loss(c, buf, acc):
        for q in range(4):
            valid = full if q < 3 else (lane < TLEN - 48)
            gidx = lane + (c * TLEN + q * LANES)
            rows = lane + q * LANES
            ix16 = plsc.load_gather(idx_v, [gidx], mask=valid)
            tg16 = plsc.load_gather(tgt_v, [gidx], mask=valid)
            tv = plsc.load_gather(buf, [rows, tg16], mask=valid)
            lsev = plsc.load_gather(lse_v, [ix16], mask=valid)
            acc = acc + jnp.where(valid, lsev - tv, 0.0)
        return acc

    start(0, buf0, sem0)

    def pair(g, acc):
        c = 2 * g
        start(c + 1, buf1, sem1)
        wait(buf0, sem0)
        acc = chunk_loss(c, buf0, acc)
        pltpu.sync_copy(buf0, out_hbm.at[bbase + c])

        @pl.when(g < N_PAIR - 1)
        def _():
            start(c + 2, buf0, sem0)

        wait(buf1, sem1)
        acc = chunk_loss(c + 1, buf1, acc)
        pltpu.sync_copy(buf1, out_hbm.at[bbase + c + 1])
        return acc

    acc = lax.fori_loop(0, N_PAIR, pair, jnp.zeros((LANES,), jnp.float32))
    part_v[...] = acc
    pltpu.sync_copy(part_v, part_hbm.at[pl.ds(wid * LANES, LANES)])


_sc_gather = functools.partial(
    pl.kernel,
    out_type=(
        jax.ShapeDtypeStruct((BATCH, TLEN, VOCAB), jnp.float32),
        jax.ShapeDtypeStruct((NW * LANES,), jnp.float32),
    ),
    mesh=plsc.VectorSubcoreMesh(
        core_axis_name="c", subcore_axis_name="s", num_cores=NC, num_subcores=NS
    ),
    scratch_types=[
        pltpu.VMEM((PAD_W,), jnp.int32),
        pltpu.VMEM((B_PER_W, TLEN), jnp.int32),
        pltpu.VMEM((PAD_W,), jnp.int32),
        pltpu.VMEM((VOCAB,), jnp.float32),
        pltpu.VMEM((LANES,), jnp.float32),
        pltpu.VMEM((TLEN, VOCAB), jnp.float32),
        pltpu.VMEM((TLEN, VOCAB), jnp.float32),
        pltpu.SemaphoreType.DMA,
        pltpu.SemaphoreType.DMA,
    ],
    compiler_params=pltpu.CompilerParams(
        use_tc_tiling_on_sc=False, needs_layout_passes=False
    ),
)(_gather_body)


def _final_body(part_ref, loss_ref):
    loss_ref[0, 0] = jnp.sum(part_ref[...]) / N_TOK


def _tc_final(partials):
    return pl.pallas_call(
        _final_body,
        out_specs=pl.BlockSpec(memory_space=pltpu.SMEM),
        out_shape=jax.ShapeDtypeStruct((1, 1), jnp.float32),
    )(partials)


def kernel(idx, target, table):
    idx_flat = idx.reshape(N_TOK).astype(jnp.int32)
    idx3 = idx_flat.reshape(NW, B_PER_W, TLEN)
    tgt_flat = target.reshape(N_TOK).astype(jnp.int32)
    lse = _tc_lse(table).reshape(VOCAB)
    logits, partials = _sc_gather(idx_flat, idx3, tgt_flat, lse, table)
    loss = _tc_final(partials.reshape(NW, LANES))
    return logits, loss[0, 0]


# speedup vs baseline: 1.0408x; 1.0408x over previous
import functools

import jax
import jax.numpy as jnp
from jax import lax
from jax.experimental import pallas as pl
from jax.experimental.pallas import tpu as pltpu
from jax.experimental.pallas import tpu_sc as plsc

VOCAB = 1000
BATCH, TLEN = 1024, 50
NC, NS = 2, 16
NW = NC * NS
N_TOK = BATCH * TLEN
B_PER_W = BATCH // NW
PER_W = B_PER_W * TLEN
N_PAIR = B_PER_W // 2
LANES = 16
PAD_W = PER_W + 64


def _lse_body(tbl_ref, lse_ref):
    l = tbl_ref[...]
    m = jnp.max(l, axis=1, keepdims=True)
    lse_ref[...] = m + jnp.log(jnp.sum(jnp.exp(l - m), axis=1, keepdims=True))


def _tc_lse(table):
    return pl.pallas_call(
        _lse_body,
        out_shape=jax.ShapeDtypeStruct((VOCAB, 1), jnp.float32),
    )(table)


def _gather_body(idx_hbm, idx3_hbm, tgt_hbm, lse_hbm, table_hbm,
                 out_hbm, part_hbm,
                 idx_v, idx2_v, tgt_v, lse_v, part_v, buf0, buf1, sem0, sem1):
    wid = lax.axis_index("s") * NC + lax.axis_index("c")
    base = wid * PER_W
    bbase = wid * B_PER_W
    pltpu.sync_copy(idx_hbm.at[pl.ds(base, PER_W)], idx_v.at[pl.ds(0, PER_W)])
    pltpu.sync_copy(tgt_hbm.at[pl.ds(base, PER_W)], tgt_v.at[pl.ds(0, PER_W)])
    pltpu.sync_copy(idx3_hbm.at[wid], idx2_v)
    pltpu.sync_copy(lse_hbm, lse_v)

    def start(c, buf, sem):
        pltpu.async_copy(table_hbm.at[idx2_v.at[c]], buf, sem)

    def wait(buf, sem):
        pltpu.make_async_copy(table_hbm.at[idx2_v.at[0]], buf, sem).wait()

    lane = lax.iota(jnp.int32, LANES)
    full = lane < LANES

    def chunk_loss(c, buf, acc):
        for q in range(4):
            valid = full if q < 3 else (lane < TLEN - 48)
            gidx = lane + (c * TLEN + q * LANES)
            rows = lane + q * LANES
            ix16 = plsc.load_gather(idx_v, [gidx], mask=valid)
            tg16 = plsc.load_gather(tgt_v, [gidx], mask=valid)
            tv = plsc.load_gather(buf, [rows, tg16], mask=valid)
            lsev = plsc.load_gather(lse_v, [ix16], mask=valid)
            acc = acc + jnp.where(valid, lsev - tv, 0.0)
        return acc

    start(0, buf0, sem0)

    def pair(g, acc):
        c = 2 * g
        start(c + 1, buf1, sem1)
        wait(buf0, sem0)
        acc = chunk_loss(c, buf0, acc)
        pltpu.sync_copy(buf0, out_hbm.at[bbase + c])

        @pl.when(g < N_PAIR - 1)
        def _():
            start(c + 2, buf0, sem0)

        wait(buf1, sem1)
        acc = chunk_loss(c + 1, buf1, acc)
        pltpu.sync_copy(buf1, out_hbm.at[bbase + c + 1])
        return acc

    acc = lax.fori_loop(0, N_PAIR, pair, jnp.zeros((LANES,), jnp.float32))
    part_v[...] = acc
    pltpu.sync_copy(part_v, part_hbm.at[pl.ds(wid * LANES, LANES)])


_sc_gather = functools.partial(
    pl.kernel,
    out_type=(
        jax.ShapeDtypeStruct((BATCH, TLEN, VOCAB), jnp.float32),
        jax.ShapeDtypeStruct((NW * LANES,), jnp.float32),
    ),
    mesh=plsc.VectorSubcoreMesh(
        core_axis_name="c", subcore_axis_name="s", num_cores=NC, num_subcores=NS
    ),
    scratch_types=[
        pltpu.VMEM((PAD_W,), jnp.int32),
        pltpu.VMEM((B_PER_W, TLEN), jnp.int32),
        pltpu.VMEM((PAD_W,), jnp.int32),
        pltpu.VMEM((VOCAB,), jnp.float32),
        pltpu.VMEM((LANES,), jnp.float32),
        pltpu.VMEM((TLEN, VOCAB), jnp.float32),
        pltpu.VMEM((TLEN, VOCAB), jnp.float32),
        pltpu.SemaphoreType.DMA,
        pltpu.SemaphoreType.DMA,
    ],
    compiler_params=pltpu.CompilerParams(
        use_tc_tiling_on_sc=False, needs_layout_passes=False
    ),
)(_gather_body)


def _final_body(part_ref, loss_ref):
    loss_ref[0, 0] = jnp.sum(part_ref[...]) / N_TOK


def _tc_final(partials):
    return pl.pallas_call(
        _final_body,
        out_specs=pl.BlockSpec(memory_space=pltpu.SMEM),
        out_shape=jax.ShapeDtypeStruct((1, 1), jnp.float32),
    )(partials)


def kernel(idx, target, table):
    idx_flat = idx.reshape(N_TOK).astype(jnp.int32)
    idx3 = idx_flat.reshape(NW, B_PER_W, TLEN)
    tgt_flat = target.reshape(N_TOK).astype(jnp.int32)
    lse = _tc_lse(table).reshape(VOCAB)
    logits, partials = _sc_gather(idx_flat, idx3, tgt_flat, lse, table)
    loss = _tc_final(partials.reshape(NW, LANES))
    return logits, loss[0, 0]
